# baseline (speedup 1.0000x reference)
import functools

import jax
import jax.numpy as jnp
from jax import lax
from jax.experimental import pallas as pl
from jax.experimental.pallas import tpu as pltpu
from jax.experimental.pallas import tpu_sc as plsc

EMB_DIM = 64
BATCH = 16384
NC, NS, LANES = 2, 16, 16
NW = NC * NS
BPW = BATCH // NW
GCHUNK = 128
NCHUNK = BPW // GCHUNK


def _sc_scores(u, v, U, V):
    mesh = plsc.VectorSubcoreMesh(core_axis_name="c", subcore_axis_name="s")

    @functools.partial(
        pl.kernel,
        out_type=jax.ShapeDtypeStruct((BATCH,), jnp.float32),
        mesh=mesh,
        scratch_types=[
            pltpu.VMEM((NCHUNK, GCHUNK), jnp.int32),
            pltpu.VMEM((NCHUNK, GCHUNK), jnp.int32),
            pltpu.VMEM((BPW, EMB_DIM), jnp.float32),
            pltpu.VMEM((BPW, EMB_DIM), jnp.float32),
            pltpu.VMEM((BPW,), jnp.float32),
            pltpu.SemaphoreType.DMA,
        ],
        compiler_params=pltpu.CompilerParams(use_tc_tiling_on_sc=False),
    )
    def scores_kernel(u_hbm, v_hbm, U_hbm, V_hbm, out_hbm,
                      idx_u, idx_v, rows_u, rows_v, scores, sem):
        wid = lax.axis_index("s") * NC + lax.axis_index("c")
        base = wid * BPW

        for j in range(NCHUNK):
            pltpu.sync_copy(u_hbm.at[pl.ds(base + j * GCHUNK, GCHUNK)],
                            idx_u.at[j])
            pltpu.sync_copy(v_hbm.at[pl.ds(base + j * GCHUNK, GCHUNK)],
                            idx_v.at[j])

        copies = []
        for j in range(NCHUNK):
            copies.append(pltpu.async_copy(
                U_hbm.at[idx_u.at[j]],
                rows_u.at[pl.ds(j * GCHUNK, GCHUNK)], sem))
            copies.append(pltpu.async_copy(
                V_hbm.at[idx_v.at[j]],
                rows_v.at[pl.ds(j * GCHUNK, GCHUNK)], sem))
        for c in copies:
            c.wait()

        lane = lax.iota(jnp.int32, LANES)

        def hsum(x):
            for s in (1, 2, 4, 8):
                x = x + jnp.take_along_axis(x, lane ^ s, axis=0)
            return x

        def group_body(g, _):
            vec = jnp.zeros((LANES,), jnp.float32)
            for k in range(LANES):
                i = g * LANES + k
                acc = rows_u[i, pl.ds(0, LANES)] * rows_v[i, pl.ds(0, LANES)]
                for cix in range(1, EMB_DIM // LANES):
                    acc = acc + (rows_u[i, pl.ds(cix * LANES, LANES)]
                                 * rows_v[i, pl.ds(cix * LANES, LANES)])
                vec = jnp.where(lane == k, hsum(acc), vec)
            scores[pl.ds(g * LANES, LANES)] = vec
            return _

        lax.fori_loop(0, BPW // LANES, group_body, 0)
        pltpu.sync_copy(scores, out_hbm.at[pl.ds(base, BPW)])

    return scores_kernel(u, v, U, V)


def _tail_kernel(s_ref, o_ref):
    x = jnp.clip(s_ref[...], -10.0, 10.0)
    o_ref[0, 0] = jnp.sum(jnp.log1p(jnp.exp(-x))) * (1.0 / BATCH)


def _tc_tail(scores):
    out = pl.pallas_call(
        _tail_kernel,
        out_shape=jax.ShapeDtypeStruct((1, 1), jnp.float32),
        in_specs=[pl.BlockSpec(memory_space=pltpu.VMEM)],
        out_specs=pl.BlockSpec(memory_space=pltpu.SMEM),
    )(scores.reshape(128, BATCH // 128))
    return out[0, 0]


def kernel(u, v, U, V):
    u = u.astype(jnp.int32)
    v = v.astype(jnp.int32)
    scores = _sc_scores(u, v, U, V)
    return _tc_tail(scores)
